# baseline (speedup 1.0000x reference)
import functools

import jax
import jax.numpy as jnp
from jax import lax
from jax.experimental import pallas as pl
from jax.experimental.pallas import tpu as pltpu
from jax.experimental.pallas import tpu_sc as plsc

EMBED_DIM = 64
BATCH = 4096
SEQ_LEN = 500

B = BATCH * SEQ_LEN
B2 = B // 2
NUM_WORKERS = 32
P_PER_W = B2 // NUM_WORKERS
P = 320
GSEG = 160
N_GSEG = P // GSEG
NBUF = 2
N_OUTER = P_PER_W // (P * NBUF)


def _sc_gather(tok_e, tok_o, table):
    mesh = plsc.VectorSubcoreMesh(core_axis_name="c", subcore_axis_name="s")

    @functools.partial(
        pl.kernel,
        mesh=mesh,
        out_type=jax.ShapeDtypeStruct((B2, 2 * EMBED_DIM), jnp.float32),
        scratch_types=[
            [pltpu.VMEM((P,), jnp.int32)] * NBUF,
            [pltpu.VMEM((P,), jnp.int32)] * NBUF,
            [pltpu.VMEM((P, EMBED_DIM), jnp.float32)] * NBUF,
            [pltpu.VMEM((P, EMBED_DIM), jnp.float32)] * NBUF,
            [pltpu.SemaphoreType.DMA] * NBUF,
            [pltpu.SemaphoreType.DMA] * NBUF,
        ],
        compiler_params=pltpu.CompilerParams(use_tc_tiling_on_sc=False),
    )
    def k(te_hbm, to_hbm, table_hbm, out_hbm,
          idxe_v, idxo_v, rowse_v, rowso_v, sem_g, sem_o):
        wid = lax.axis_index("s") * 2 + lax.axis_index("c")
        w_base = wid * P_PER_W

        def body(t, carry):
            for b in range(NBUF):
                base = w_base + (t * NBUF + b) * P

                @pl.when(t > 0)
                def _drain_prev_write(b=b):
                    pltpu.make_async_copy(
                        rowse_v[b],
                        out_hbm.at[pl.ds(0, P), pl.ds(0, EMBED_DIM)],
                        sem_o[b],
                    ).wait()
                    pltpu.make_async_copy(
                        rowso_v[b],
                        out_hbm.at[pl.ds(0, P), pl.ds(EMBED_DIM, EMBED_DIM)],
                        sem_o[b],
                    ).wait()

                pltpu.sync_copy(te_hbm.at[pl.ds(base, P)], idxe_v[b])
                pltpu.sync_copy(to_hbm.at[pl.ds(base, P)], idxo_v[b])
                for j in range(N_GSEG):
                    s = pl.ds(j * GSEG, GSEG)
                    pltpu.async_copy(
                        table_hbm.at[idxe_v[b].at[s]],
                        rowse_v[b].at[s],
                        sem_g[b],
                    )
                    pltpu.async_copy(
                        table_hbm.at[idxo_v[b].at[s]],
                        rowso_v[b].at[s],
                        sem_g[b],
                    )
            for b in range(NBUF):
                base = w_base + (t * NBUF + b) * P
                for j in range(N_GSEG):
                    s = pl.ds(j * GSEG, GSEG)
                    pltpu.make_async_copy(
                        table_hbm.at[idxe_v[b].at[s]],
                        rowse_v[b].at[s],
                        sem_g[b],
                    ).wait()
                    pltpu.make_async_copy(
                        table_hbm.at[idxo_v[b].at[s]],
                        rowso_v[b].at[s],
                        sem_g[b],
                    ).wait()
                pltpu.async_copy(
                    rowse_v[b],
                    out_hbm.at[pl.ds(base, P), pl.ds(0, EMBED_DIM)],
                    sem_o[b],
                )
                pltpu.async_copy(
                    rowso_v[b],
                    out_hbm.at[pl.ds(base, P), pl.ds(EMBED_DIM, EMBED_DIM)],
                    sem_o[b],
                )
            return carry

        lax.fori_loop(0, N_OUTER, body, 0)
        for b in range(NBUF):
            pltpu.make_async_copy(
                rowse_v[b],
                out_hbm.at[pl.ds(0, P), pl.ds(0, EMBED_DIM)],
                sem_o[b],
            ).wait()
            pltpu.make_async_copy(
                rowso_v[b],
                out_hbm.at[pl.ds(0, P), pl.ds(EMBED_DIM, EMBED_DIM)],
                sem_o[b],
            ).wait()

    return k(tok_e, tok_o, table)


def kernel(tokens, table):
    tok2 = tokens.reshape(B2, 2)
    out2 = _sc_gather(tok2[:, 0], tok2[:, 1], table)
    return out2.reshape(BATCH, SEQ_LEN, EMBED_DIM)


# speedup vs baseline: 1.2509x; 1.2509x over previous
import functools

import jax
import jax.numpy as jnp
from jax import lax
from jax.experimental import pallas as pl
from jax.experimental.pallas import tpu as pltpu
from jax.experimental.pallas import tpu_sc as plsc

EMBED_DIM = 64
BATCH = 4096
SEQ_LEN = 500

B = BATCH * SEQ_LEN
NUM_WORKERS = 32
BATCH_PER_W = BATCH // NUM_WORKERS
BPC = 2
CHUNK = BPC * SEQ_LEN
GATHER = 200
N_GATHER = CHUNK // GATHER
NBUF = 2
N_OUTER = BATCH_PER_W // (BPC * NBUF)


def _sc_gather(tokens_flat, table):
    mesh = plsc.VectorSubcoreMesh(core_axis_name="c", subcore_axis_name="s")

    @functools.partial(
        pl.kernel,
        mesh=mesh,
        out_type=jax.ShapeDtypeStruct((BATCH, SEQ_LEN, EMBED_DIM), jnp.float32),
        scratch_types=[
            [pltpu.VMEM((CHUNK,), jnp.int32)] * NBUF,
            [pltpu.VMEM((CHUNK, EMBED_DIM), jnp.float32)] * NBUF,
            [pltpu.SemaphoreType.DMA] * NBUF,
            [pltpu.SemaphoreType.DMA] * NBUF,
        ],
        compiler_params=pltpu.CompilerParams(use_tc_tiling_on_sc=False),
    )
    def k(tok_hbm, table_hbm, out_hbm, idx_v, rows_v, sem_g, sem_o):
        wid = lax.axis_index("s") * 2 + lax.axis_index("c")
        w_batch = wid * BATCH_PER_W

        def body(t, carry):
            for b in range(NBUF):
                batch0 = w_batch + (t * NBUF + b) * BPC

                @pl.when(t > 0)
                def _drain_prev_writes(b=b):
                    for p in range(BPC):
                        pltpu.make_async_copy(
                            rows_v[b].at[pl.ds(p * SEQ_LEN, SEQ_LEN)],
                            out_hbm.at[0],
                            sem_o[b],
                        ).wait()

                pltpu.sync_copy(
                    tok_hbm.at[pl.ds(batch0 * SEQ_LEN, CHUNK)], idx_v[b]
                )
                for j in range(N_GATHER):
                    pltpu.async_copy(
                        table_hbm.at[idx_v[b].at[pl.ds(j * GATHER, GATHER)]],
                        rows_v[b].at[pl.ds(j * GATHER, GATHER)],
                        sem_g[b],
                    )
            for b in range(NBUF):
                batch0 = w_batch + (t * NBUF + b) * BPC
                for j in range(N_GATHER):
                    pltpu.make_async_copy(
                        table_hbm.at[idx_v[b].at[pl.ds(j * GATHER, GATHER)]],
                        rows_v[b].at[pl.ds(j * GATHER, GATHER)],
                        sem_g[b],
                    ).wait()
                for p in range(BPC):
                    pltpu.async_copy(
                        rows_v[b].at[pl.ds(p * SEQ_LEN, SEQ_LEN)],
                        out_hbm.at[batch0 + p],
                        sem_o[b],
                    )
            return carry

        lax.fori_loop(0, N_OUTER, body, 0)
        for b in range(NBUF):
            for p in range(BPC):
                pltpu.make_async_copy(
                    rows_v[b].at[pl.ds(p * SEQ_LEN, SEQ_LEN)],
                    out_hbm.at[0],
                    sem_o[b],
                ).wait()

    return k(tokens_flat, table)


def kernel(tokens, table):
    return _sc_gather(tokens.reshape(B), table)


# speedup vs baseline: 1.2758x; 1.0199x over previous
import functools

import jax
import jax.numpy as jnp
from jax import lax
from jax.experimental import pallas as pl
from jax.experimental.pallas import tpu as pltpu
from jax.experimental.pallas import tpu_sc as plsc

EMBED_DIM = 64
BATCH = 4096
SEQ_LEN = 500

NUM_WORKERS = 32
NPART = 4
PART_BATCH = BATCH // NPART
BATCH_PER_W = PART_BATCH // NUM_WORKERS
BPC = 2
CHUNK = BPC * SEQ_LEN
GATHER = 200
N_GATHER = CHUNK // GATHER
NBUF = 2
N_OUTER = BATCH_PER_W // (BPC * NBUF)


def _sc_gather(tokens_flat, table):
    mesh = plsc.VectorSubcoreMesh(core_axis_name="c", subcore_axis_name="s")

    @functools.partial(
        pl.kernel,
        mesh=mesh,
        out_type=jax.ShapeDtypeStruct(
            (PART_BATCH, SEQ_LEN, EMBED_DIM), jnp.float32
        ),
        scratch_types=[
            [pltpu.VMEM((CHUNK,), jnp.int32)] * NBUF,
            [pltpu.VMEM((CHUNK, EMBED_DIM), jnp.float32)] * NBUF,
            [pltpu.SemaphoreType.DMA] * NBUF,
            [pltpu.SemaphoreType.DMA] * NBUF,
        ],
        compiler_params=pltpu.CompilerParams(use_tc_tiling_on_sc=False),
    )
    def k(tok_hbm, table_hbm, out_hbm, idx_v, rows_v, sem_g, sem_o):
        wid = lax.axis_index("s") * 2 + lax.axis_index("c")
        w_batch = wid * BATCH_PER_W

        def body(t, carry):
            for b in range(NBUF):
                batch0 = w_batch + (t * NBUF + b) * BPC

                @pl.when(t > 0)
                def _drain_prev_writes(b=b):
                    for p in range(BPC):
                        pltpu.make_async_copy(
                            rows_v[b].at[pl.ds(p * SEQ_LEN, SEQ_LEN)],
                            out_hbm.at[0],
                            sem_o[b],
                        ).wait()

                pltpu.sync_copy(
                    tok_hbm.at[pl.ds(batch0 * SEQ_LEN, CHUNK)], idx_v[b]
                )
                for j in range(N_GATHER):
                    pltpu.async_copy(
                        table_hbm.at[idx_v[b].at[pl.ds(j * GATHER, GATHER)]],
                        rows_v[b].at[pl.ds(j * GATHER, GATHER)],
                        sem_g[b],
                    )
            for b in range(NBUF):
                batch0 = w_batch + (t * NBUF + b) * BPC
                for j in range(N_GATHER):
                    pltpu.make_async_copy(
                        table_hbm.at[idx_v[b].at[pl.ds(j * GATHER, GATHER)]],
                        rows_v[b].at[pl.ds(j * GATHER, GATHER)],
                        sem_g[b],
                    ).wait()
                for p in range(BPC):
                    pltpu.async_copy(
                        rows_v[b].at[pl.ds(p * SEQ_LEN, SEQ_LEN)],
                        out_hbm.at[batch0 + p],
                        sem_o[b],
                    )
            return carry

        lax.fori_loop(0, N_OUTER, body, 0)
        for b in range(NBUF):
            for p in range(BPC):
                pltpu.make_async_copy(
                    rows_v[b].at[pl.ds(p * SEQ_LEN, SEQ_LEN)],
                    out_hbm.at[0],
                    sem_o[b],
                ).wait()

    return k(tokens_flat, table)


def kernel(tokens, table):
    tokens_flat = tokens.reshape(BATCH * SEQ_LEN)
    n = PART_BATCH * SEQ_LEN
    parts = [
        _sc_gather(lax.dynamic_slice(tokens_flat, (i * n,), (n,)), table)
        for i in range(NPART)
    ]
    return jnp.concatenate(parts, axis=0)
